# baseline (speedup 1.0000x reference)
import jax
import jax.numpy as jnp
from jax import lax
from jax.experimental import pallas as pl
from jax.experimental.pallas import tpu as pltpu

_NP = 5120
_T = 128
_NB = _NP // _T
_IOU_TH = 0.7
_SCORE_TH = 0.05


def _prep_body(coords_ref, o_ref, scores_ref,
               xyxy_ref, msc_ref, score_ref, label_ref, mask_ref, nv_ref):
    o = o_ref[...]
    cls = scores_ref[:, 0, :]
    lab = jnp.zeros(cls.shape, jnp.int32)
    for c in range(1, 20):
        v = scores_ref[:, c, :]
        better = v > cls
        cls = jnp.where(better, v, cls)
        lab = jnp.where(better, c, lab)
    score = cls * o
    mask = (o >= 0.5) & (score >= _SCORE_TH)
    x = coords_ref[:, 0, :]
    y = coords_ref[:, 1, :]
    w = coords_ref[:, 2, :]
    h = coords_ref[:, 3, :]
    xyxy_ref[:, 0, :] = jnp.clip(x - w / 2.0, 0.0, 1.0)
    xyxy_ref[:, 1, :] = jnp.clip(y - h / 2.0, 0.0, 1.0)
    xyxy_ref[:, 2, :] = jnp.clip(x + w / 2.0, 0.0, 1.0)
    xyxy_ref[:, 3, :] = jnp.clip(y + h / 2.0, 0.0, 1.0)
    score_ref[...] = score
    label_ref[...] = lab
    mask_ref[...] = mask.astype(jnp.int32)
    msc_ref[...] = jnp.where(mask, score, -jnp.inf)
    nv_ref[...] = jnp.sum(mask.astype(jnp.int32), axis=1, keepdims=True)


def _nms_body(nv_ref, rows_ref, keep_ref, cols_s):
    keep_ref[0] = jnp.ones((_NB, _T), jnp.float32)
    eye = (lax.broadcasted_iota(jnp.int32, (_T, _T), 0) ==
           lax.broadcasted_iota(jnp.int32, (_T, _T), 1)).astype(jnp.float32)
    rowlt = (lax.broadcasted_iota(jnp.int32, (_T, _T), 0) <
             lax.broadcasted_iota(jnp.int32, (_T, _T), 1))

    nv = nv_ref[pl.program_id(0)]
    nbv = (nv + _T - 1) // _T

    def get_row(c, j):
        return rows_ref[0, c, pl.ds(j * _T, _T)].reshape(1, _T)

    def to_col(v):
        return lax.dot_general(eye, v, (((1,), (1,)), ((), ())),
                               preferred_element_type=jnp.float32)

    def build_cols(j, _):
        x1c = to_col(get_row(0, j))
        y1c = to_col(get_row(1, j))
        x2c = to_col(get_row(2, j))
        y2c = to_col(get_row(3, j))
        base = j * _T
        cols_s[pl.ds(base, _T), 0:1] = x1c
        cols_s[pl.ds(base, _T), 1:2] = y1c
        cols_s[pl.ds(base, _T), 2:3] = x2c
        cols_s[pl.ds(base, _T), 3:4] = y2c
        cols_s[pl.ds(base, _T), 4:5] = (x2c - x1c) * (y2c - y1c)
        return 0

    lax.fori_loop(0, nbv, build_cols, 0)

    def over_j(j, _):
        xj1 = get_row(0, j)
        yj1 = get_row(1, j)
        xj2 = get_row(2, j)
        yj2 = get_row(3, j)
        area_j = (xj2 - xj1) * (yj2 - yj1)

        def iou_tile(a):
            base = a * _T
            xa1 = cols_s[pl.ds(base, _T), 0:1]
            ya1 = cols_s[pl.ds(base, _T), 1:2]
            xa2 = cols_s[pl.ds(base, _T), 2:3]
            ya2 = cols_s[pl.ds(base, _T), 3:4]
            area_a = cols_s[pl.ds(base, _T), 4:5]
            iw = jnp.maximum(
                jnp.minimum(xa2, xj2) - jnp.maximum(xa1, xj1), 0.0)
            ih = jnp.maximum(
                jnp.minimum(ya2, yj2) - jnp.maximum(ya1, yj1), 0.0)
            inter = iw * ih
            return inter / (area_a + area_j - inter + 1e-12)

        def over_a(a, sup):
            iou = iou_tile(a)
            ka = cols_s[pl.ds(a * _T, _T), 5:6]
            hit = jnp.where(iou > _IOU_TH, ka, 0.0)
            return jnp.maximum(sup, jnp.max(hit, axis=0, keepdims=True))

        sup = lax.fori_loop(0, j, over_a, jnp.zeros((1, _T), jnp.float32))
        kv0 = 1.0 - sup

        sm = jnp.where((iou_tile(j) > _IOU_TH) & rowlt, 1.0, 0.0)

        def fstep(kv):
            sup_d = jnp.max(sm * to_col(kv), axis=0, keepdims=True)
            return kv0 * (1.0 - sup_d)

        kv1 = fstep(kv0)

        def fcond(st):
            kv, kprev = st
            return jnp.any(kv != kprev)

        def fbody(st):
            kv, _ = st
            return (fstep(kv), kv)

        kv, _ = lax.while_loop(fcond, fbody, (kv1, kv0))
        keep_ref[0, pl.ds(j, 1), :] = kv
        cols_s[pl.ds(j * _T, _T), 5:6] = to_col(kv)
        return 0

    lax.fori_loop(0, nbv, over_j, 0)


@jax.jit
def kernel(b_coords, b_o, b_scores):
    B, N, C = b_scores.shape
    pad = _NP - N
    coords_t = jnp.pad(jnp.transpose(b_coords, (0, 2, 1)),
                       ((0, 0), (0, 0), (0, pad)))
    o_p = jnp.pad(b_o, ((0, 0), (0, pad)))
    scores_t = jnp.pad(jnp.transpose(b_scores, (0, 2, 1)),
                       ((0, 0), (0, 0), (0, pad)))

    xyxy_t, msc, score, lab, mask, nv = pl.pallas_call(
        _prep_body,
        out_shape=[
            jax.ShapeDtypeStruct((B, 4, _NP), jnp.float32),
            jax.ShapeDtypeStruct((B, _NP), jnp.float32),
            jax.ShapeDtypeStruct((B, _NP), jnp.float32),
            jax.ShapeDtypeStruct((B, _NP), jnp.int32),
            jax.ShapeDtypeStruct((B, _NP), jnp.int32),
            jax.ShapeDtypeStruct((B, 1), jnp.int32),
        ],
    )(coords_t, o_p, scores_t)

    order = jnp.argsort(-msc, axis=-1)
    bs = jnp.take_along_axis(xyxy_t, order[:, None, :], axis=2)

    keep_s = pl.pallas_call(
        _nms_body,
        grid_spec=pltpu.PrefetchScalarGridSpec(
            num_scalar_prefetch=1,
            grid=(B,),
            in_specs=[
                pl.BlockSpec((1, 4, _NP), lambda b, nv_s: (b, 0, 0)),
            ],
            out_specs=pl.BlockSpec((1, _NB, _T), lambda b, nv_s: (b, 0, 0)),
            scratch_shapes=[pltpu.VMEM((_NP, 128), jnp.float32)],
        ),
        out_shape=jax.ShapeDtypeStruct((B, _NB, _T), jnp.float32),
    )(nv.reshape(B), bs)

    keep_sorted = keep_s.reshape(B, _NP) > 0.5
    keep = jnp.zeros((B, _NP), bool).at[
        jnp.arange(B)[:, None], order].set(keep_sorted)
    final = (mask > 0) & keep
    final = final[:, :N]
    xyxy = jnp.transpose(xyxy_t, (0, 2, 1))[:, :N, :]
    boxes_out = xyxy * final[..., None].astype(xyxy.dtype)
    scores_out = jnp.where(final, score[:, :N], 0.0)
    labels_out = jnp.where(final, lab[:, :N], -1)
    return boxes_out, scores_out, labels_out, final


# speedup vs baseline: 1.3064x; 1.3064x over previous
import jax
import jax.numpy as jnp
from jax import lax
from jax.experimental import pallas as pl
from jax.experimental.pallas import tpu as pltpu

_NP = 5120
_T = 128
_NB = _NP // _T
_IOU_TH = 0.7
_SCORE_TH = 0.05


def _prep_body(coords_ref, o_ref, scores_ref,
               xyxy_ref, msc_ref, score_ref, label_ref, mask_ref, nv_ref):
    o = o_ref[...]
    cls = scores_ref[:, 0, :]
    lab = jnp.zeros(cls.shape, jnp.int32)
    for c in range(1, 20):
        v = scores_ref[:, c, :]
        better = v > cls
        cls = jnp.where(better, v, cls)
        lab = jnp.where(better, c, lab)
    score = cls * o
    mask = (o >= 0.5) & (score >= _SCORE_TH)
    x = coords_ref[:, 0, :]
    y = coords_ref[:, 1, :]
    w = coords_ref[:, 2, :]
    h = coords_ref[:, 3, :]
    xyxy_ref[:, 0, :] = jnp.clip(x - w / 2.0, 0.0, 1.0)
    xyxy_ref[:, 1, :] = jnp.clip(y - h / 2.0, 0.0, 1.0)
    xyxy_ref[:, 2, :] = jnp.clip(x + w / 2.0, 0.0, 1.0)
    xyxy_ref[:, 3, :] = jnp.clip(y + h / 2.0, 0.0, 1.0)
    score_ref[...] = score
    label_ref[...] = lab
    mask_ref[...] = mask.astype(jnp.int32)
    msc_ref[...] = jnp.where(mask, score, -jnp.inf)
    nv_ref[...] = jnp.sum(mask.astype(jnp.int32), axis=1, keepdims=True)


def _nms_body(nv_ref, rows_ref, keep_ref, cols_s):
    keep_ref[0] = jnp.ones((_NB, _T), jnp.float32)
    eye = (lax.broadcasted_iota(jnp.int32, (_T, _T), 0) ==
           lax.broadcasted_iota(jnp.int32, (_T, _T), 1)).astype(jnp.float32)
    rowlt = (lax.broadcasted_iota(jnp.int32, (_T, _T), 0) <
             lax.broadcasted_iota(jnp.int32, (_T, _T), 1))

    nv = nv_ref[pl.program_id(0)]
    nbv = (nv + _T - 1) // _T

    def get_row(c, j):
        return rows_ref[0, c, pl.ds(j * _T, _T)].reshape(1, _T)

    def to_col(v):
        return lax.dot_general(eye, v, (((1,), (1,)), ((), ())),
                               preferred_element_type=jnp.float32)

    def build_cols(j, _):
        x1c = to_col(get_row(0, j))
        y1c = to_col(get_row(1, j))
        x2c = to_col(get_row(2, j))
        y2c = to_col(get_row(3, j))
        base = j * _T
        cols_s[pl.ds(base, _T), 0:1] = x1c
        cols_s[pl.ds(base, _T), 1:2] = y1c
        cols_s[pl.ds(base, _T), 2:3] = x2c
        cols_s[pl.ds(base, _T), 3:4] = y2c
        cols_s[pl.ds(base, _T), 4:5] = (x2c - x1c) * (y2c - y1c)
        return 0

    lax.fori_loop(0, nbv, build_cols, 0)

    def over_j(j, _):
        xj1 = get_row(0, j)
        yj1 = get_row(1, j)
        xj2 = get_row(2, j)
        yj2 = get_row(3, j)
        area_j = (xj2 - xj1) * (yj2 - yj1)

        def iou_tile(a):
            base = a * _T
            xa1 = cols_s[pl.ds(base, _T), 0:1]
            ya1 = cols_s[pl.ds(base, _T), 1:2]
            xa2 = cols_s[pl.ds(base, _T), 2:3]
            ya2 = cols_s[pl.ds(base, _T), 3:4]
            area_a = cols_s[pl.ds(base, _T), 4:5]
            iw = jnp.maximum(
                jnp.minimum(xa2, xj2) - jnp.maximum(xa1, xj1), 0.0)
            ih = jnp.maximum(
                jnp.minimum(ya2, yj2) - jnp.maximum(ya1, yj1), 0.0)
            inter = iw * ih
            return inter / (area_a + area_j - inter + 1e-12)

        def over_a(a, sup):
            iou = iou_tile(a)
            ka = cols_s[pl.ds(a * _T, _T), 5:6]
            hit = jnp.where(iou > _IOU_TH, ka, 0.0)
            return jnp.maximum(sup, jnp.max(hit, axis=0, keepdims=True))

        sup = lax.fori_loop(0, j, over_a, jnp.zeros((1, _T), jnp.float32))
        kv0 = 1.0 - sup

        sm = jnp.where((iou_tile(j) > _IOU_TH) & rowlt, 1.0, 0.0)

        def fstep(kv):
            sup_d = jnp.max(sm * to_col(kv), axis=0, keepdims=True)
            return kv0 * (1.0 - sup_d)

        kv1 = fstep(kv0)

        def fcond(st):
            kv, kprev = st
            return jnp.any(kv != kprev)

        def fbody(st):
            kv, _ = st
            return (fstep(kv), kv)

        kv, _ = lax.while_loop(fcond, fbody, (kv1, kv0))
        keep_ref[0, pl.ds(j, 1), :] = kv
        cols_s[pl.ds(j * _T, _T), 5:6] = to_col(kv)
        return 0

    lax.fori_loop(0, nbv, over_j, 0)


@jax.jit
def kernel(b_coords, b_o, b_scores):
    B, N, C = b_scores.shape
    pad = _NP - N
    coords_t = jnp.pad(jnp.transpose(b_coords, (0, 2, 1)),
                       ((0, 0), (0, 0), (0, pad)))
    o_p = jnp.pad(b_o, ((0, 0), (0, pad)))
    scores_t = jnp.pad(jnp.transpose(b_scores, (0, 2, 1)),
                       ((0, 0), (0, 0), (0, pad)))

    xyxy_t, msc, score, lab, mask, nv = pl.pallas_call(
        _prep_body,
        out_shape=[
            jax.ShapeDtypeStruct((B, 4, _NP), jnp.float32),
            jax.ShapeDtypeStruct((B, _NP), jnp.float32),
            jax.ShapeDtypeStruct((B, _NP), jnp.float32),
            jax.ShapeDtypeStruct((B, _NP), jnp.int32),
            jax.ShapeDtypeStruct((B, _NP), jnp.int32),
            jax.ShapeDtypeStruct((B, 1), jnp.int32),
        ],
    )(coords_t, o_p, scores_t)

    order = jnp.argsort(-msc, axis=-1)
    bs = jnp.take_along_axis(xyxy_t, order[:, None, :], axis=2)

    keep_s = pl.pallas_call(
        _nms_body,
        grid_spec=pltpu.PrefetchScalarGridSpec(
            num_scalar_prefetch=1,
            grid=(B,),
            in_specs=[
                pl.BlockSpec((1, 4, _NP), lambda b, nv_s: (b, 0, 0)),
            ],
            out_specs=pl.BlockSpec((1, _NB, _T), lambda b, nv_s: (b, 0, 0)),
            scratch_shapes=[pltpu.VMEM((_NP, 128), jnp.float32)],
        ),
        out_shape=jax.ShapeDtypeStruct((B, _NB, _T), jnp.float32),
    )(nv.reshape(B), bs)

    keep_sorted = keep_s.reshape(B, _NP) > 0.5
    inv_order = jnp.argsort(order, axis=-1)
    keep = jnp.take_along_axis(keep_sorted, inv_order, axis=1)
    final = (mask > 0) & keep
    final = final[:, :N]
    xyxy = jnp.transpose(xyxy_t, (0, 2, 1))[:, :N, :]
    boxes_out = xyxy * final[..., None].astype(xyxy.dtype)
    scores_out = jnp.where(final, score[:, :N], 0.0)
    labels_out = jnp.where(final, lab[:, :N], -1)
    return boxes_out, scores_out, labels_out, final


# speedup vs baseline: 1.8019x; 1.3792x over previous
import jax
import jax.numpy as jnp
from jax import lax
from jax.experimental import pallas as pl
from jax.experimental.pallas import tpu as pltpu

_NP = 5120
_W = 256
_NBW = _NP // _W
_IOU_TH = 0.7
_SCORE_TH = 0.05


def _prep_body(coords_ref, o_ref, scores_ref,
               xyxy_ref, msc_ref, score_ref, label_ref, mask_ref, nv_ref):
    o = o_ref[...]
    cls = scores_ref[:, 0, :]
    lab = jnp.zeros(cls.shape, jnp.int32)
    for c in range(1, 20):
        v = scores_ref[:, c, :]
        better = v > cls
        cls = jnp.where(better, v, cls)
        lab = jnp.where(better, c, lab)
    score = cls * o
    mask = (o >= 0.5) & (score >= _SCORE_TH)
    x = coords_ref[:, 0, :]
    y = coords_ref[:, 1, :]
    w = coords_ref[:, 2, :]
    h = coords_ref[:, 3, :]
    xyxy_ref[:, 0, :] = jnp.clip(x - w / 2.0, 0.0, 1.0)
    xyxy_ref[:, 1, :] = jnp.clip(y - h / 2.0, 0.0, 1.0)
    xyxy_ref[:, 2, :] = jnp.clip(x + w / 2.0, 0.0, 1.0)
    xyxy_ref[:, 3, :] = jnp.clip(y + h / 2.0, 0.0, 1.0)
    score_ref[...] = score
    label_ref[...] = lab
    mask_ref[...] = mask.astype(jnp.int32)
    msc_ref[...] = jnp.where(mask, score, -jnp.inf)
    nv_ref[...] = jnp.sum(mask.astype(jnp.int32), axis=1, keepdims=True)


def _nms_body(nv_ref, rows_ref, keep_ref, cols_s):
    keep_ref[0] = jnp.ones((_NBW, _W), jnp.float32)
    eye = (lax.broadcasted_iota(jnp.int32, (_W, _W), 0) ==
           lax.broadcasted_iota(jnp.int32, (_W, _W), 1)).astype(jnp.float32)
    rowlt = (lax.broadcasted_iota(jnp.int32, (_W, _W), 0) <
             lax.broadcasted_iota(jnp.int32, (_W, _W), 1))

    nv = nv_ref[pl.program_id(0)]
    nbv = (nv + _W - 1) // _W

    def get_row(c, j):
        return rows_ref[0, c, pl.ds(j * _W, _W)].reshape(1, _W)

    def to_col(v):
        return lax.dot_general(eye, v, (((1,), (1,)), ((), ())),
                               preferred_element_type=jnp.float32)

    def build_cols(j, _):
        x1c = to_col(get_row(0, j))
        y1c = to_col(get_row(1, j))
        x2c = to_col(get_row(2, j))
        y2c = to_col(get_row(3, j))
        base = j * _W
        cols_s[pl.ds(base, _W), 0:1] = x1c
        cols_s[pl.ds(base, _W), 1:2] = y1c
        cols_s[pl.ds(base, _W), 2:3] = x2c
        cols_s[pl.ds(base, _W), 3:4] = y2c
        cols_s[pl.ds(base, _W), 4:5] = (x2c - x1c) * (y2c - y1c)
        return 0

    lax.fori_loop(0, nbv, build_cols, 0)

    def over_j(j, _):
        xj1 = get_row(0, j)
        yj1 = get_row(1, j)
        xj2 = get_row(2, j)
        yj2 = get_row(3, j)
        area_j = (xj2 - xj1) * (yj2 - yj1)

        def s_tile(a):
            base = a * _W
            xa1 = cols_s[pl.ds(base, _W), 0:1]
            ya1 = cols_s[pl.ds(base, _W), 1:2]
            xa2 = cols_s[pl.ds(base, _W), 2:3]
            ya2 = cols_s[pl.ds(base, _W), 3:4]
            area_a = cols_s[pl.ds(base, _W), 4:5]
            iw = jnp.maximum(
                jnp.minimum(xa2, xj2) - jnp.maximum(xa1, xj1), 0.0)
            ih = jnp.maximum(
                jnp.minimum(ya2, yj2) - jnp.maximum(ya1, yj1), 0.0)
            inter = iw * ih
            iou = inter / (area_a + area_j - inter + 1e-12)
            return jnp.where(iou > _IOU_TH, 1.0, 0.0).astype(jnp.bfloat16)

        def over_a(a, cnt):
            s = s_tile(a)
            ka = keep_ref[0, pl.ds(a, 1), :].astype(jnp.bfloat16)
            return cnt + lax.dot_general(
                ka, s, (((1,), (0,)), ((), ())),
                preferred_element_type=jnp.float32)

        cnt = lax.fori_loop(0, j, over_a, jnp.zeros((1, _W), jnp.float32))
        kv0 = jnp.where(cnt > 0.0, 0.0, 1.0)

        sm = jnp.where(rowlt, s_tile(j), jnp.bfloat16(0.0))

        def fstep(kv):
            cnt_d = lax.dot_general(
                kv.astype(jnp.bfloat16), sm, (((1,), (0,)), ((), ())),
                preferred_element_type=jnp.float32)
            return jnp.where(cnt_d > 0.0, 0.0, kv0)

        kv1 = fstep(kv0)

        def fcond(st):
            kv, kprev = st
            return jnp.any(kv != kprev)

        def fbody(st):
            kv, _ = st
            return (fstep(kv), kv)

        kv, _ = lax.while_loop(fcond, fbody, (kv1, kv0))
        keep_ref[0, pl.ds(j, 1), :] = kv
        return 0

    lax.fori_loop(0, nbv, over_j, 0)


@jax.jit
def kernel(b_coords, b_o, b_scores):
    B, N, C = b_scores.shape
    pad = _NP - N
    coords_t = jnp.pad(jnp.transpose(b_coords, (0, 2, 1)),
                       ((0, 0), (0, 0), (0, pad)))
    o_p = jnp.pad(b_o, ((0, 0), (0, pad)))
    scores_t = jnp.pad(jnp.transpose(b_scores, (0, 2, 1)),
                       ((0, 0), (0, 0), (0, pad)))

    xyxy_t, msc, score, lab, mask, nv = pl.pallas_call(
        _prep_body,
        out_shape=[
            jax.ShapeDtypeStruct((B, 4, _NP), jnp.float32),
            jax.ShapeDtypeStruct((B, _NP), jnp.float32),
            jax.ShapeDtypeStruct((B, _NP), jnp.float32),
            jax.ShapeDtypeStruct((B, _NP), jnp.int32),
            jax.ShapeDtypeStruct((B, _NP), jnp.int32),
            jax.ShapeDtypeStruct((B, 1), jnp.int32),
        ],
    )(coords_t, o_p, scores_t)

    order = jnp.argsort(-msc, axis=-1)
    bs = jnp.take_along_axis(xyxy_t, order[:, None, :], axis=2)

    keep_s = pl.pallas_call(
        _nms_body,
        grid_spec=pltpu.PrefetchScalarGridSpec(
            num_scalar_prefetch=1,
            grid=(B,),
            in_specs=[
                pl.BlockSpec((1, 4, _NP), lambda b, nv_s: (b, 0, 0)),
            ],
            out_specs=pl.BlockSpec((1, _NBW, _W), lambda b, nv_s: (b, 0, 0)),
            scratch_shapes=[pltpu.VMEM((_NP, 128), jnp.float32)],
        ),
        out_shape=jax.ShapeDtypeStruct((B, _NBW, _W), jnp.float32),
    )(nv.reshape(B), bs)

    keep_sorted = keep_s.reshape(B, _NP) > 0.5
    inv_order = jnp.argsort(order, axis=-1)
    keep = jnp.take_along_axis(keep_sorted, inv_order, axis=1)
    final = (mask > 0) & keep
    final = final[:, :N]
    xyxy = jnp.transpose(xyxy_t, (0, 2, 1))[:, :N, :]
    boxes_out = xyxy * final[..., None].astype(xyxy.dtype)
    scores_out = jnp.where(final, score[:, :N], 0.0)
    labels_out = jnp.where(final, lab[:, :N], -1)
    return boxes_out, scores_out, labels_out, final
